# baseline (speedup 1.0000x reference)
import functools

import jax
import jax.numpy as jnp
from jax import lax
from jax.experimental import pallas as pl
from jax.experimental.pallas import tpu as pltpu
from jax.experimental.pallas import tpu_sc as plsc

NC = 2
NS = 16
NW = NC * NS

NB = 16384
F = 26
D = 32
DP = 128
VOC = 1000000
IPW = NB // NW
IG = 8
NG = IPW // IG


@functools.cache
def _build_gather_kernel():
    mesh = plsc.VectorSubcoreMesh(core_axis_name="c", subcore_axis_name="s")

    @functools.partial(
        pl.kernel,
        out_type=jax.ShapeDtypeStruct((NB, 32, 128), jnp.float32),
        mesh=mesh,
        compiler_params=pltpu.CompilerParams(use_tc_tiling_on_sc=False),
        scratch_types=[
            pltpu.VMEM((IPW, F), jnp.int32),
            pltpu.VMEM((2, IG, F, DP), jnp.float32),
            pltpu.SemaphoreType.DMA,
            pltpu.SemaphoreType.DMA,
            pltpu.SemaphoreType.DMA,
            pltpu.SemaphoreType.DMA,
        ],
    )
    def gather_kernel(idx_hbm, table_hbm, out_hbm, idx_v, rows_v,
                      gsem0, gsem1, wsem0, wsem1):
        gsems = (gsem0, gsem1)
        wsems = (wsem0, wsem1)
        wid = lax.axis_index("s") * NC + lax.axis_index("c")
        pltpu.sync_copy(idx_hbm.at[pl.ds(wid * IPW, IPW)], idx_v)
        i_base0 = wid * IPW

        def fire(g, buf):
            for l in range(IG):
                pltpu.async_copy(
                    table_hbm.at[idx_v.at[g * IG + l]],
                    rows_v.at[buf].at[l],
                    gsems[buf],
                )

        def drain_gathers(buf):
            pltpu.make_async_copy(
                out_hbm.at[pl.ds(0, IG), pl.ds(0, F), pl.ds(0, DP)],
                rows_v.at[buf],
                gsems[buf],
            ).wait()

        def start_write(g, buf):
            pltpu.async_copy(
                rows_v.at[buf].at[pl.ds(0, IG), pl.ds(0, F), pl.ds(0, D)],
                out_hbm.at[pl.ds(i_base0 + g * IG, IG), pl.ds(0, F),
                           pl.ds(0, D)],
                wsems[buf],
            )

        def wait_write(buf):
            pltpu.make_async_copy(
                rows_v.at[buf].at[pl.ds(0, IG), pl.ds(0, F), pl.ds(0, D)],
                out_hbm.at[pl.ds(0, IG), pl.ds(0, F), pl.ds(0, D)],
                wsems[buf],
            ).wait()

        def step(g, buf, first=False, last=False):
            if not last:
                if not first:
                    wait_write(1 - buf)
                fire(g + 1, 1 - buf)
            drain_gathers(buf)
            start_write(g, buf)

        fire(0, 0)
        step(0, 0, first=True)
        pl.loop(1, NG - 1, step=2, unroll=1)(
            lambda g: (step(g, 1), step(g + 1, 0)) and None
        )
        step(NG - 1, 1, last=True)

        wait_write(0)
        wait_write(1)

    return gather_kernel


def kernel(indices, parameter):
    table_pad = jnp.pad(parameter, ((0, 0), (0, DP - D)))
    out = _build_gather_kernel()(indices, table_pad)
    return out[:, :F, :D]


# speedup vs baseline: 1.0874x; 1.0874x over previous
import functools

import jax
import jax.numpy as jnp
from jax import lax
from jax.experimental import pallas as pl
from jax.experimental.pallas import tpu as pltpu
from jax.experimental.pallas import tpu_sc as plsc

NC = 2
NS = 16
NW = NC * NS

NB = 16384
F = 26
D = 32
DP = 128
VOC = 1000000
IPW = NB // NW
IG = 8
NG = IPW // IG


@functools.cache
def _build_gather_kernel():
    mesh = plsc.VectorSubcoreMesh(core_axis_name="c", subcore_axis_name="s")

    @functools.partial(
        pl.kernel,
        out_type=jax.ShapeDtypeStruct((NB, 32, 128), jnp.float32),
        mesh=mesh,
        compiler_params=pltpu.CompilerParams(use_tc_tiling_on_sc=False),
        scratch_types=[
            pltpu.VMEM((IPW, F), jnp.int32),
            pltpu.VMEM((2, IG, F, D), jnp.float32),
            pltpu.SemaphoreType.DMA,
            pltpu.SemaphoreType.DMA,
            pltpu.SemaphoreType.DMA,
            pltpu.SemaphoreType.DMA,
        ],
    )
    def gather_kernel(idx_hbm, table_hbm, out_hbm, idx_v, rows_v,
                      gsem0, gsem1, wsem0, wsem1):
        gsems = (gsem0, gsem1)
        wsems = (wsem0, wsem1)
        wid = lax.axis_index("s") * NC + lax.axis_index("c")
        pltpu.sync_copy(idx_hbm.at[pl.ds(wid * IPW, IPW)], idx_v)
        i_base0 = wid * IPW

        def fire(g, buf):
            for l in range(IG):
                pltpu.async_copy(
                    table_hbm.at[idx_v.at[g * IG + l]],
                    rows_v.at[buf].at[l],
                    gsems[buf],
                )

        def drain_gathers(buf):
            pltpu.make_async_copy(
                out_hbm.at[pl.ds(0, IG), pl.ds(0, F), pl.ds(0, D)],
                rows_v.at[buf],
                gsems[buf],
            ).wait()

        def start_write(g, buf):
            pltpu.async_copy(
                rows_v.at[buf],
                out_hbm.at[pl.ds(i_base0 + g * IG, IG), pl.ds(0, F),
                           pl.ds(0, D)],
                wsems[buf],
            )

        def wait_write(buf):
            pltpu.make_async_copy(
                rows_v.at[buf],
                out_hbm.at[pl.ds(0, IG), pl.ds(0, F), pl.ds(0, D)],
                wsems[buf],
            ).wait()

        def step(g, buf, first=False, last=False):
            if not last:
                if not first:
                    wait_write(1 - buf)
                fire(g + 1, 1 - buf)
            drain_gathers(buf)
            start_write(g, buf)

        fire(0, 0)
        step(0, 0, first=True)
        pl.loop(1, NG - 1, step=2, unroll=1)(
            lambda g: (step(g, 1), step(g + 1, 0)) and None
        )
        step(NG - 1, 1, last=True)

        wait_write(0)
        wait_write(1)

    return gather_kernel


def kernel(indices, parameter):
    table_view = jnp.pad(parameter, ((0, 0), (0, DP - D))).reshape(4 * VOC, D)
    out = _build_gather_kernel()(indices * 4, table_view)
    return out[:, :F, :D]
